# baseline (speedup 1.0000x reference)
import functools

import jax
import jax.numpy as jnp
from jax import lax
from jax.experimental import pallas as pl
from jax.experimental.pallas import tpu as pltpu
from jax.experimental.pallas import tpu_sc as plsc

NC = 2
NS = 16
LCH = 128


def _make_agg(n, d, n_pad, ch):
  rows_per_tile = n_pad // NS
  chp = ch // 2
  mesh = plsc.VectorSubcoreMesh(
      core_axis_name="c", subcore_axis_name="s",
      num_cores=NC, num_subcores=NS)

  @functools.partial(
      pl.kernel,
      out_type=jax.ShapeDtypeStruct((NC, n_pad, d), jnp.float32),
      mesh=mesh,
      scratch_types=[
          pltpu.VMEM((chp, LCH), jnp.int32),
          pltpu.VMEM((chp, LCH), jnp.int32),
          pltpu.VMEM((LCH, d), jnp.float32),
          pltpu.VMEM((LCH, d), jnp.float32),
          pltpu.VMEM_SHARED((n_pad, d), jnp.float32),
          pltpu.SemaphoreType.DMA,
          pltpu.SemaphoreType.DMA,
      ],
  )
  def agg_kernel(h_hbm, srcp_hbm, dstp_hbm, zeros_hbm, out_hbm,
                 src_idx, dst_idx, buf_a, buf_b, acc, sem_a, sem_b):
    c = lax.axis_index("c")
    s = lax.axis_index("s")
    wid = (1 - c) * NS + s

    pltpu.sync_copy(zeros_hbm,
                    acc.at[pl.ds(s * rows_per_tile, rows_per_tile)])
    plsc.subcore_barrier()

    def gather(j, buf, sem):
      pltpu.async_copy(h_hbm.at[src_idx.at[j]], buf, sem)

    def wait(buf, sem):
      pltpu.make_async_copy(h_hbm.at[pl.ds(0, LCH)], buf, sem).wait()

    def scatter_add(j, buf):
      pltpu.sync_copy(buf, acc.at[dst_idx.at[j]], add=True)

    @pl.loop(0, 2)
    def _(p):
      pltpu.sync_copy(srcp_hbm.at[wid, pl.ds(p * chp, chp)], src_idx)
      pltpu.sync_copy(dstp_hbm.at[wid, pl.ds(p * chp, chp)], dst_idx)

      gather(0, buf_a, sem_a)

      @pl.loop(0, chp, step=2)
      def _(g):
        gather(g + 1, buf_b, sem_b)
        wait(buf_a, sem_a)
        scatter_add(g, buf_a)

        @pl.when(g + 2 < chp)
        def _():
          gather(g + 2, buf_a, sem_a)

        wait(buf_b, sem_b)
        scatter_add(g + 1, buf_b)

    plsc.subcore_barrier()
    pltpu.sync_copy(acc.at[pl.ds(s * rows_per_tile, rows_per_tile)],
                    out_hbm.at[c, pl.ds(s * rows_per_tile, rows_per_tile)])

  return agg_kernel


def _mlp1_body(x_ref, p_ref, w1_ref, b1_ref, w2_ref, b2_ref, o_ref):
  z = x_ref[...] + p_ref[0] + p_ref[1]
  t = jnp.dot(z, w1_ref[...], preferred_element_type=jnp.float32)
  t = jnp.maximum(t + b1_ref[...], 0.0)
  h = jnp.dot(t, w2_ref[...], preferred_element_type=jnp.float32)
  o_ref[...] = jnp.maximum(h + b2_ref[...], 0.0)


def _mlp2_body(h_ref, q_ref, w1_ref, b1_ref, w2_ref, b2_ref,
               wc_ref, bc_ref, o_ref):
  z = h_ref[...] + q_ref[0] + q_ref[1]
  t = jnp.dot(z, w1_ref[...], preferred_element_type=jnp.float32)
  t = jnp.maximum(t + b1_ref[...], 0.0)
  h2 = jnp.dot(t, w2_ref[...], preferred_element_type=jnp.float32)
  h2 = jnp.maximum(h2 + b2_ref[...], 0.0)
  o = jnp.dot(h2, wc_ref[...], preferred_element_type=jnp.float32)
  o_ref[...] = o + bc_ref[...]


def _full_spec(shape):
  return pl.BlockSpec(shape, lambda i: (0,) * len(shape))


def _mlp1_call(x, p, w1, b1, w2, b2, bm):
  n, d = x.shape
  h = w1.shape[1]
  grid = (n // bm,)
  return pl.pallas_call(
      _mlp1_body,
      grid=grid,
      in_specs=[
          pl.BlockSpec((bm, d), lambda i: (i, 0)),
          pl.BlockSpec((NC, bm, d), lambda i: (0, i, 0)),
          _full_spec(w1.shape),
          _full_spec((1, h)),
          _full_spec(w2.shape),
          _full_spec((1, h)),
      ],
      out_specs=pl.BlockSpec((bm, h), lambda i: (i, 0)),
      out_shape=jax.ShapeDtypeStruct((n, h), jnp.float32),
  )(x, p, w1, b1.reshape(1, -1), w2, b2.reshape(1, -1))


def _mlp2_call(hh, q, w1, b1, w2, b2, wc, bc, bm):
  n, d = hh.shape
  h = w1.shape[1]
  c = wc.shape[1]
  grid = (n // bm,)
  return pl.pallas_call(
      _mlp2_body,
      grid=grid,
      in_specs=[
          pl.BlockSpec((bm, d), lambda i: (i, 0)),
          pl.BlockSpec((NC, bm, d), lambda i: (0, i, 0)),
          _full_spec(w1.shape),
          _full_spec((1, h)),
          _full_spec(w2.shape),
          _full_spec((1, h)),
          _full_spec(wc.shape),
          _full_spec((1, c)),
      ],
      out_specs=pl.BlockSpec((bm, c), lambda i: (i, 0)),
      out_shape=jax.ShapeDtypeStruct((n, c), jnp.float32),
  )(hh, q, w1, b1.reshape(1, -1), w2, b2.reshape(1, -1),
    wc, bc.reshape(1, -1))


def kernel(x, edge_index, W11, b11, W12, b12, W21, b21, W22, b22, Wc, bc):
  n, d = x.shape
  e = edge_index.shape[1]
  nw = NC * NS
  ch = -(-e // (nw * LCH))
  ch = -(-ch // 4) * 4
  e_pad = nw * ch * LCH

  ei = edge_index.astype(jnp.int32)
  src = jnp.concatenate([ei[0], jnp.zeros((e_pad - e,), jnp.int32)])
  dst = jnp.concatenate([ei[1], jnp.full((e_pad - e,), n, jnp.int32)])
  srcp = src.reshape(nw, ch, LCH)
  dstp = dst.reshape(nw, ch, LCH)
  n_pad = -(-(n + 1) // (NS * 8)) * (NS * 8)
  zeros = jnp.zeros((n_pad // NS, d), jnp.float32)

  agg = _make_agg(n, d, n_pad, ch)
  bm = 2000

  p1 = agg(x, srcp, dstp, zeros)
  h1 = _mlp1_call(x, p1, W11, b11, W12, b12, bm)
  p2 = agg(h1, srcp, dstp, zeros)
  return _mlp2_call(h1, p2, W21, b21, W22, b22, Wc, bc, bm)


# speedup vs baseline: 1.2643x; 1.2643x over previous
import functools

import jax
import jax.numpy as jnp
from jax import lax
from jax.experimental import pallas as pl
from jax.experimental.pallas import tpu as pltpu
from jax.experimental.pallas import tpu_sc as plsc

NC = 2
NS = 16
LCH = 128


def _make_agg(n, d, n_pad, ch):
  rows_per_tile = n_pad // NS
  chp = ch // 2
  mesh = plsc.VectorSubcoreMesh(
      core_axis_name="c", subcore_axis_name="s",
      num_cores=NC, num_subcores=NS)

  @functools.partial(
      pl.kernel,
      out_type=jax.ShapeDtypeStruct((NC, n_pad, d), jnp.float32),
      mesh=mesh,
      scratch_types=[
          pltpu.VMEM((chp, LCH), jnp.int32),
          pltpu.VMEM((chp, LCH), jnp.int32),
          pltpu.VMEM((LCH, d), jnp.float32),
          pltpu.VMEM((LCH, d), jnp.float32),
          pltpu.VMEM_SHARED((n_pad, d), jnp.float32),
          pltpu.SemaphoreType.DMA,
          pltpu.SemaphoreType.DMA,
      ],
  )
  def agg_kernel(h_hbm, srcp_hbm, dstp_hbm, zeros_hbm, out_hbm,
                 src_idx, dst_idx, buf_a, buf_b, acc, sem_a, sem_b):
    c = lax.axis_index("c")
    s = lax.axis_index("s")
    wid = c * NS + s

    pltpu.sync_copy(zeros_hbm,
                    acc.at[pl.ds(s * rows_per_tile, rows_per_tile)])
    plsc.subcore_barrier()

    def gather(j, buf, sem):
      pltpu.async_copy(h_hbm.at[src_idx.at[j]], buf, sem)

    def wait(buf, sem):
      pltpu.make_async_copy(h_hbm.at[pl.ds(0, LCH)], buf, sem).wait()

    def scatter_add(j, buf):
      pltpu.sync_copy(buf, acc.at[dst_idx.at[j]], add=True)

    @pl.loop(0, 2)
    def _(p):
      pltpu.sync_copy(srcp_hbm.at[wid, pl.ds(p * chp, chp)], src_idx)
      pltpu.sync_copy(dstp_hbm.at[wid, pl.ds(p * chp, chp)], dst_idx)

      gather(0, buf_a, sem_a)

      @pl.loop(0, chp, step=2)
      def _(g):
        gather(g + 1, buf_b, sem_b)
        wait(buf_a, sem_a)
        scatter_add(g, buf_a)

        @pl.when(g + 2 < chp)
        def _():
          gather(g + 2, buf_a, sem_a)

        wait(buf_b, sem_b)
        scatter_add(g + 1, buf_b)

    plsc.subcore_barrier()
    pltpu.sync_copy(acc.at[pl.ds(s * rows_per_tile, rows_per_tile)],
                    out_hbm.at[c, pl.ds(s * rows_per_tile, rows_per_tile)])

  return agg_kernel


def _mlp1_body(x_ref, p_ref, w1_ref, b1_ref, w2_ref, b2_ref, o_ref):
  z = x_ref[...] + p_ref[0] + p_ref[1]
  t = jnp.dot(z, w1_ref[...], preferred_element_type=jnp.float32)
  t = jnp.maximum(t + b1_ref[...], 0.0)
  h = jnp.dot(t, w2_ref[...], preferred_element_type=jnp.float32)
  o_ref[...] = jnp.maximum(h + b2_ref[...], 0.0)


def _mlp2_body(h_ref, q_ref, w1_ref, b1_ref, w2_ref, b2_ref,
               wc_ref, bc_ref, o_ref):
  z = h_ref[...] + q_ref[0] + q_ref[1]
  t = jnp.dot(z, w1_ref[...], preferred_element_type=jnp.float32)
  t = jnp.maximum(t + b1_ref[...], 0.0)
  h2 = jnp.dot(t, w2_ref[...], preferred_element_type=jnp.float32)
  h2 = jnp.maximum(h2 + b2_ref[...], 0.0)
  o = jnp.dot(h2, wc_ref[...], preferred_element_type=jnp.float32)
  o_ref[...] = o + bc_ref[...]


def _full_spec(shape):
  return pl.BlockSpec(shape, lambda i: (0,) * len(shape))


def _mlp1_call(x, p, w1, b1, w2, b2, bm):
  n, d = x.shape
  h = w1.shape[1]
  grid = (n // bm,)
  return pl.pallas_call(
      _mlp1_body,
      grid=grid,
      in_specs=[
          pl.BlockSpec((bm, d), lambda i: (i, 0)),
          pl.BlockSpec((NC, bm, d), lambda i: (0, i, 0)),
          _full_spec(w1.shape),
          _full_spec((1, h)),
          _full_spec(w2.shape),
          _full_spec((1, h)),
      ],
      out_specs=pl.BlockSpec((bm, h), lambda i: (i, 0)),
      out_shape=jax.ShapeDtypeStruct((n, h), jnp.float32),
  )(x, p, w1, b1.reshape(1, -1), w2, b2.reshape(1, -1))


def _mlp2_call(hh, q, w1, b1, w2, b2, wc, bc, bm):
  n, d = hh.shape
  h = w1.shape[1]
  c = wc.shape[1]
  grid = (n // bm,)
  return pl.pallas_call(
      _mlp2_body,
      grid=grid,
      in_specs=[
          pl.BlockSpec((bm, d), lambda i: (i, 0)),
          pl.BlockSpec((NC, bm, d), lambda i: (0, i, 0)),
          _full_spec(w1.shape),
          _full_spec((1, h)),
          _full_spec(w2.shape),
          _full_spec((1, h)),
          _full_spec(wc.shape),
          _full_spec((1, c)),
      ],
      out_specs=pl.BlockSpec((bm, c), lambda i: (i, 0)),
      out_shape=jax.ShapeDtypeStruct((n, c), jnp.float32),
  )(hh, q, w1, b1.reshape(1, -1), w2, b2.reshape(1, -1),
    wc, bc.reshape(1, -1))


def kernel(x, edge_index, W11, b11, W12, b12, W21, b21, W22, b22, Wc, bc):
  n, d = x.shape
  e = edge_index.shape[1]
  nw = NC * NS
  ch = -(-e // (nw * LCH))
  ch = -(-ch // 4) * 4
  e_pad = nw * ch * LCH

  n_pad = -(-(n + 1) // (NS * 8)) * (NS * 8)

  ei = edge_index.astype(jnp.int32)
  e_tile = -(-e // nw)
  ei = jnp.pad(ei, ((0, 0), (0, nw * e_tile - e)))
  pad_per_tile = ch * LCH - e_tile
  pad_src = jnp.zeros((nw, pad_per_tile), jnp.int32)
  pad_dst = jnp.broadcast_to(
      n + (jnp.arange(pad_per_tile, dtype=jnp.int32) % (n_pad - n)),
      (nw, pad_per_tile))
  real_valid = jnp.arange(nw * e_tile, dtype=jnp.int32).reshape(nw, e_tile) < e
  src2 = ei[0].reshape(nw, e_tile)
  dst2 = jnp.where(real_valid, ei[1].reshape(nw, e_tile), pad_dst[:, :1])
  srcp = jnp.concatenate([src2, pad_src], axis=1).reshape(nw, ch, LCH)
  dstp = jnp.concatenate([dst2, pad_dst], axis=1).reshape(nw, ch, LCH)
  zeros = jnp.zeros((n_pad // NS, d), jnp.float32)

  agg = _make_agg(n, d, n_pad, ch)
  bm = 2000

  p1 = agg(x, srcp, dstp, zeros)
  h1 = _mlp1_call(x, p1, W11, b11, W12, b12, bm)
  p2 = agg(h1, srcp, dstp, zeros)
  return _mlp2_call(h1, p2, W21, b21, W22, b22, Wc, bc, bm)
